# baseline (speedup 1.0000x reference)
import jax
import jax.numpy as jnp
from jax import lax
from jax.experimental import pallas as pl
from jax.experimental.pallas import tpu as pltpu
from jax.experimental.pallas import tpu_sc as plsc

B = 4096
L = 200
D = 128
V = 44
VP = 48
NT = B * L
LBLK = 40

SPLIT = 409600
TK = 1600

NWORK = 32
PER_TILE = (NT - SPLIT) // NWORK
C = 256
NC = PER_TILE // C


def _fused_table_body(table_ref, out_ref, pe_ref):
    i = pl.program_id(0)
    d = lax.broadcasted_iota(jnp.int32, (LBLK, 1, D), 2)
    pos = (lax.broadcasted_iota(jnp.int32, (LBLK, 1, D), 0) + i * LBLK)
    posf = pos.astype(jnp.float32)
    half = (d // 2).astype(jnp.float32)
    denom = jnp.exp(half * (2.0 / D) * jnp.log(10000.0))
    ang = posf / denom
    pe = jnp.where(d % 2 == 0, jnp.sin(ang), jnp.cos(ang))
    tab = table_ref[...]
    tab = jnp.concatenate([tab, jnp.zeros((VP - V, D), jnp.float32)], axis=0)
    out_ref[...] = tab[None, :, :] + pe
    pe_ref[...] = pe[:, 0, :]


def _flat_idx_body(x_ref, out_ref):
    l = lax.broadcasted_iota(jnp.int32, x_ref.shape, 1)
    out_ref[...] = x_ref[...] + VP * l


def _build_fused_and_pe(table):
    return pl.pallas_call(
        _fused_table_body,
        grid=(L // LBLK,),
        in_specs=[pl.BlockSpec((V, D), lambda i: (0, 0))],
        out_specs=[
            pl.BlockSpec((LBLK, VP, D), lambda i: (i, 0, 0)),
            pl.BlockSpec((LBLK, D), lambda i: (i, 0)),
        ],
        out_shape=[
            jax.ShapeDtypeStruct((L, VP, D), jnp.float32),
            jax.ShapeDtypeStruct((L, D), jnp.float32),
        ],
    )(table)


def _build_flat_idx(x):
    return pl.pallas_call(
        _flat_idx_body,
        grid=(8,),
        in_specs=[pl.BlockSpec((B // 8, L), lambda i: (i, 0))],
        out_specs=pl.BlockSpec((B // 8, L), lambda i: (i, 0)),
        out_shape=jax.ShapeDtypeStruct((B, L), jnp.int32),
    )(x)


def _sc_gather(fused, idx):
    mesh = plsc.VectorSubcoreMesh(
        core_axis_name="core", subcore_axis_name="subcore")

    @pl.kernel(
        out_type=jax.ShapeDtypeStruct((NT, D), jnp.float32),
        mesh=mesh,
        scratch_types=[
            pltpu.VMEM((PER_TILE,), jnp.int32),
            pltpu.VMEM((C, D), jnp.float32),
            pltpu.VMEM((C, D), jnp.float32),
            pltpu.SemaphoreType.DMA,
            pltpu.SemaphoreType.DMA,
            pltpu.SemaphoreType.DMA,
            pltpu.SemaphoreType.DMA,
        ],
    )
    def gather_kernel(fused_hbm, idx_hbm, out_hbm,
                      idx_v, buf_a, buf_b, sg_a, sg_b, sw_a, sw_b):
        wid = lax.axis_index("subcore") * 2 + lax.axis_index("core")
        base = SPLIT + wid * PER_TILE
        bufs = (buf_a, buf_b)
        gsems = (sg_a, sg_b)
        wsems = (sw_a, sw_b)

        pltpu.sync_copy(idx_hbm.at[pl.ds(base, PER_TILE)], idx_v)

        def gather_copy(cc, b):
            return pltpu.make_async_copy(
                fused_hbm.at[idx_v.at[pl.ds(cc * C, C)]], bufs[b], gsems[b])

        gather_copy(0, 0).start()
        gather_copy(1, 1).start()

        @pl.loop(0, NC, step=2)
        def _(c):
            for b in range(2):
                cc = c + b
                gather_copy(cc, b).wait()
                wb = pltpu.make_async_copy(
                    bufs[b], out_hbm.at[pl.ds(base + cc * C, C)], wsems[b])
                wb.start()
                wb.wait()

                @pl.when(cc + 2 < NC)
                def _():
                    gather_copy(cc + 2, b).start()

    return gather_kernel(fused, idx.reshape(NT))


def _tc_merge_body(x_ref, tab_ref, pe_ref, alias_ref, out_ref):
    xb = x_ref[...]
    oh = (xb == lax.broadcasted_iota(jnp.int32, (TK, V), 1)).astype(
        jnp.float32)
    rows = jnp.dot(oh, tab_ref[...], preferred_element_type=jnp.float32)
    pe = jnp.concatenate([pe_ref[...]] * (TK // L), axis=0)
    out_ref[...] = rows + pe


def _tc_merge(x_front, table, pe, sc_full):
    return pl.pallas_call(
        _tc_merge_body,
        grid=(SPLIT // TK,),
        in_specs=[
            pl.BlockSpec((TK, 1), lambda i: (i, 0)),
            pl.BlockSpec((V, D), lambda i: (0, 0)),
            pl.BlockSpec((L, D), lambda i: (0, 0)),
            pl.BlockSpec((8, D), lambda i: (0, 0)),
        ],
        out_specs=pl.BlockSpec((TK, D), lambda i: (i, 0)),
        out_shape=jax.ShapeDtypeStruct((NT, D), jnp.float32),
        input_output_aliases={3: 0},
    )(x_front, table, pe, sc_full)


def kernel(x, start_token, end_token, table):
    fused, pe = _build_fused_and_pe(table)
    fused = fused.reshape(L * VP, D)
    idx = _build_flat_idx(x)
    sc_full = _sc_gather(fused, idx)
    x_front = x.reshape(NT)[:SPLIT].reshape(SPLIT, 1)
    out = _tc_merge(x_front, table, pe, sc_full)
    return out.reshape(B, L, D)


# speedup vs baseline: 1.4867x; 1.4867x over previous
import jax
import jax.numpy as jnp
from jax import lax
from jax.experimental import pallas as pl
from jax.experimental.pallas import tpu as pltpu
from jax.experimental.pallas import tpu_sc as plsc

B = 4096
L = 200
D = 128
V = 44
VP = 48
NT = B * L
LBLK = 40

NWORK = 32
PER_TILE = NT // NWORK
C = 128
NB = 4
NC = PER_TILE // C


def _fused_table_body(table_ref, out_ref):
    i = pl.program_id(0)
    d = lax.broadcasted_iota(jnp.int32, (LBLK, 1, D), 2)
    pos = (lax.broadcasted_iota(jnp.int32, (LBLK, 1, D), 0) + i * LBLK)
    posf = pos.astype(jnp.float32)
    half = (d // 2).astype(jnp.float32)
    denom = jnp.exp(half * (2.0 / D) * jnp.log(10000.0))
    ang = posf / denom
    pe = jnp.where(d % 2 == 0, jnp.sin(ang), jnp.cos(ang))
    tab = table_ref[...]
    tab = jnp.concatenate([tab, jnp.zeros((VP - V, D), jnp.float32)], axis=0)
    out_ref[...] = tab[None, :, :] + pe


def _flat_idx_body(x_ref, out_ref):
    l = lax.broadcasted_iota(jnp.int32, x_ref.shape, 1)
    out_ref[...] = x_ref[...] + VP * l


def _build_fused(table):
    return pl.pallas_call(
        _fused_table_body,
        grid=(L // LBLK,),
        in_specs=[pl.BlockSpec((V, D), lambda i: (0, 0))],
        out_specs=pl.BlockSpec((LBLK, VP, D), lambda i: (i, 0, 0)),
        out_shape=jax.ShapeDtypeStruct((L, VP, D), jnp.float32),
    )(table)


def _build_flat_idx(x):
    return pl.pallas_call(
        _flat_idx_body,
        grid=(8,),
        in_specs=[pl.BlockSpec((B // 8, L), lambda i: (i, 0))],
        out_specs=pl.BlockSpec((B // 8, L), lambda i: (i, 0)),
        out_shape=jax.ShapeDtypeStruct((B, L), jnp.int32),
    )(x)


def _sc_gather(fused, idx):
    mesh = plsc.VectorSubcoreMesh(
        core_axis_name="core", subcore_axis_name="subcore")

    @pl.kernel(
        out_type=jax.ShapeDtypeStruct((NT, D), jnp.float32),
        mesh=mesh,
        scratch_types=(
            [pltpu.VMEM((PER_TILE,), jnp.int32)]
            + [pltpu.VMEM((C, D), jnp.float32) for _ in range(NB)]
            + [pltpu.SemaphoreType.DMA for _ in range(2 * NB)]
        ),
    )
    def gather_kernel(fused_hbm, idx_hbm, out_hbm, idx_v, *bufsem):
        bufs = bufsem[:NB]
        gsems = bufsem[NB:2 * NB]
        wsems = bufsem[2 * NB:]
        wid = lax.axis_index("subcore") * 2 + lax.axis_index("core")
        base = wid * PER_TILE

        pltpu.sync_copy(idx_hbm.at[pl.ds(base, PER_TILE)], idx_v)

        def gcopy(cc, b):
            return pltpu.make_async_copy(
                fused_hbm.at[idx_v.at[pl.ds(cc * C, C)]], bufs[b], gsems[b])

        def wcopy(cc, b):
            return pltpu.make_async_copy(
                bufs[b], out_hbm.at[pl.ds(base + cc * C, C)], wsems[b])

        for b in range(NB):
            gcopy(b, b).start()

        @pl.loop(0, NC, step=NB)
        def _(c):
            for b in range(NB):
                cc = c + b
                gcopy(cc, b).wait()
                wcopy(cc, b).start()
            for b in range(NB):
                cc = c + b
                wcopy(cc, b).wait()

                @pl.when(cc + NB < NC)
                def _():
                    gcopy(cc + NB, b).start()

    return gather_kernel(fused, idx.reshape(NT))


def kernel(x, start_token, end_token, table):
    fused = _build_fused(table).reshape(L * VP, D)
    idx = _build_flat_idx(x)
    out = _sc_gather(fused, idx)
    return out.reshape(B, L, D)


# speedup vs baseline: 1.9301x; 1.2983x over previous
import jax
import jax.numpy as jnp
from jax import lax
from jax.experimental import pallas as pl
from jax.experimental.pallas import tpu as pltpu
from jax.experimental.pallas import tpu_sc as plsc

B = 4096
L = 200
D = 128
V = 44
VP = 48
NT = B * L
LBLK = 40

NWORK = 32
PER_TILE = NT // NWORK
C = 80
NB = 5
NC = PER_TILE // C


def _fused_table_body(table_ref, out_ref):
    i = pl.program_id(0)
    d = lax.broadcasted_iota(jnp.int32, (LBLK, 1, D), 2)
    pos = (lax.broadcasted_iota(jnp.int32, (LBLK, 1, D), 0) + i * LBLK)
    posf = pos.astype(jnp.float32)
    half = (d // 2).astype(jnp.float32)
    denom = jnp.exp(half * (2.0 / D) * jnp.log(10000.0))
    ang = posf / denom
    pe = jnp.where(d % 2 == 0, jnp.sin(ang), jnp.cos(ang))
    tab = table_ref[...]
    tab = jnp.concatenate([tab, jnp.zeros((VP - V, D), jnp.float32)], axis=0)
    out_ref[...] = tab[None, :, :] + pe


def _flat_idx_body(x_ref, out_ref):
    l = lax.broadcasted_iota(jnp.int32, x_ref.shape, 1)
    out_ref[...] = x_ref[...] + VP * l


def _build_fused(table):
    return pl.pallas_call(
        _fused_table_body,
        grid=(L // LBLK,),
        in_specs=[pl.BlockSpec((V, D), lambda i: (0, 0))],
        out_specs=pl.BlockSpec((LBLK, VP, D), lambda i: (i, 0, 0)),
        out_shape=jax.ShapeDtypeStruct((L, VP, D), jnp.float32),
    )(table)


def _build_flat_idx(x):
    return pl.pallas_call(
        _flat_idx_body,
        grid=(8,),
        in_specs=[pl.BlockSpec((B // 8, L), lambda i: (i, 0))],
        out_specs=pl.BlockSpec((B // 8, L), lambda i: (i, 0)),
        out_shape=jax.ShapeDtypeStruct((B, L), jnp.int32),
    )(x)


def _sc_gather(fused, idx):
    mesh = plsc.VectorSubcoreMesh(
        core_axis_name="core", subcore_axis_name="subcore")

    @pl.kernel(
        out_type=jax.ShapeDtypeStruct((NT, D), jnp.float32),
        mesh=mesh,
        scratch_types=(
            [pltpu.VMEM_SHARED((L * VP, D), jnp.float32)]
            + [pltpu.VMEM((C,), jnp.int32) for _ in range(NB)]
            + [pltpu.VMEM((C, D), jnp.float32) for _ in range(NB)]
            + [pltpu.SemaphoreType.DMA for _ in range(2 * NB)]
        ),
    )
    def gather_kernel(fused_hbm, idx_hbm, out_hbm, fused_sp, *bufsem):
        ibufs = bufsem[:NB]
        bufs = bufsem[NB:2 * NB]
        gsems = bufsem[2 * NB:3 * NB]
        wsems = bufsem[3 * NB:]
        wid = lax.axis_index("subcore") * 2 + lax.axis_index("core")
        base = wid * PER_TILE

        @pl.when(lax.axis_index("subcore") == 0)
        def _():
            pltpu.sync_copy(fused_hbm, fused_sp)
        plsc.subcore_barrier()

        def istage(cc, b):
            pltpu.sync_copy(idx_hbm.at[pl.ds(base + cc * C, C)], ibufs[b])

        def gcopy(cc, b):
            return pltpu.make_async_copy(
                fused_sp.at[ibufs[b]], bufs[b], gsems[b])

        def wcopy(cc, b):
            return pltpu.make_async_copy(
                bufs[b], out_hbm.at[pl.ds(base + cc * C, C)], wsems[b])

        for b in range(NB):
            istage(b, b)
            gcopy(b, b).start()

        @pl.loop(0, NC, step=NB)
        def _(c):
            for b in range(NB):
                cc = c + b
                gcopy(cc, b).wait()
                wcopy(cc, b).start()

                @pl.when(cc + NB < NC)
                def _():
                    istage(cc + NB, b)
            for b in range(NB):
                cc = c + b
                wcopy(cc, b).wait()

                @pl.when(cc + NB < NC)
                def _():
                    gcopy(cc + NB, b).start()

    return gather_kernel(fused, idx.reshape(NT))


def kernel(x, start_token, end_token, table):
    fused = _build_fused(table).reshape(L * VP, D)
    idx = _build_flat_idx(x)
    out = _sc_gather(fused, idx)
    return out.reshape(B, L, D)
